# baseline (speedup 1.0000x reference)
import functools
import math

import jax
import jax.numpy as jnp
from jax import lax
from jax.experimental import pallas as pl
from jax.experimental.pallas import tpu as pltpu
from jax.experimental.pallas import tpu_sc as plsc

D = 256
NH = 8
HD = 32
NQ = 1024
K = 128
TOTAL = 43520
LVL_W0 = 1024
LVL_H0 = 32
SCALE = math.sqrt(HD)

SC_CORES = 2
SC_SUBCORES = 16
NW = SC_CORES * SC_SUBCORES
ROWS = NQ * K
RPW = ROWS // NW
CH = 128
NCH = RPW // CH
NBUF = 3


def _prep_kernel(q_ref, tc_ref, fc_ref, wqt_ref, bq_ref,
                 qout_ref, idx_ref, mask_ref):
    qout_ref[...] = (
        jnp.dot(q_ref[...], wqt_ref[...], preferred_element_type=jnp.float32)
        + bq_ref[...]
    )
    tc = tc_ref[...]
    fc = fc_ref[...]
    col = lax.broadcasted_iota(jnp.int32, (NQ, K), 1)
    lvl = col // 32
    within = col % 32
    t_off = within % 8 - 4
    f_off = within // 8 - 2
    w_i = jnp.int32(LVL_W0) >> lvl
    h_i = jnp.int32(LVL_H0) >> lvl
    lsi = jnp.where(lvl == 0, 0,
          jnp.where(lvl == 1, 32768,
          jnp.where(lvl == 2, 40960, 43008)))
    tpx = jnp.round(tc * w_i.astype(jnp.float32) - 0.5).astype(jnp.int32)
    fpx = jnp.round(fc * h_i.astype(jnp.float32) - 0.5).astype(jnp.int32)
    tt = tpx + t_off
    ff = fpx + f_off
    oob = (tt < 0) | (tt >= w_i) | (ff < 0) | (ff >= h_i)
    ttc = jnp.clip(tt, 0, w_i - 1)
    ffc = jnp.clip(ff, 0, h_i - 1)
    idx_ref[...] = lsi + ffc * w_i + ttc
    mask_ref[...] = jnp.where(oob, 0.0, 1.0)


def _prep(q2, tc2, fc2, wqt, bq2):
    return pl.pallas_call(
        _prep_kernel,
        out_shape=[
            jax.ShapeDtypeStruct((NQ, D), jnp.float32),
            jax.ShapeDtypeStruct((NQ, K), jnp.int32),
            jax.ShapeDtypeStruct((NQ, K), jnp.float32),
        ],
    )(q2, tc2, fc2, wqt, bq2)


DW = D // 2


def _sc_gather(value2, idx_flat):
    mesh = plsc.VectorSubcoreMesh(core_axis_name="c", subcore_axis_name="s")

    @functools.partial(
        pl.kernel,
        mesh=mesh,
        out_type=jax.ShapeDtypeStruct((ROWS, DW), jnp.int32),
        scratch_types=[
            pltpu.VMEM((RPW,), jnp.int32),
            [pltpu.VMEM((CH, DW), jnp.int32) for _ in range(NBUF)],
            [pltpu.SemaphoreType.DMA for _ in range(NBUF)],
            [pltpu.SemaphoreType.DMA for _ in range(NBUF)],
        ],
    )
    def gather(value_hbm, idx_hbm, out_hbm, idx_all, bufs, gsems, ssems):
        wid = lax.axis_index("s") * SC_CORES + lax.axis_index("c")
        base = wid * RPW
        pltpu.sync_copy(idx_hbm.at[pl.ds(base, RPW)], idx_all)

        def fire_gather(g, b):
            return pltpu.async_copy(
                value_hbm.at[idx_all.at[pl.ds(g * CH, CH)]], bufs[b], gsems[b])

        def fire_scatter(g, b):
            return pltpu.async_copy(
                bufs[b], out_hbm.at[pl.ds(base + g * CH, CH)], ssems[b])

        gd = [None] * NBUF
        sd = [None] * NBUF
        for g in range(NBUF):
            gd[g] = fire_gather(g, g)
        for g in range(NCH):
            b = g % NBUF
            gd[b].wait()
            sd[b] = fire_scatter(g, b)
            nxt = g + NBUF
            if nxt < NCH:
                sd[b].wait()
                gd[b] = fire_gather(nxt, b)
        for g in range(NCH - min(NBUF, NCH), NCH):
            sd[g % NBUF].wait()

    return gather(value2, idx_flat)


QB = 64
TB = QB * K


def _attn_kernel(toks_ref, q_ref, mask_ref, wkt_ref, wvt_ref, wot_ref,
                 bk_ref, bv_ref, bo_ref, out_ref):
    toks = toks_ref[...]
    k = (jnp.dot(toks, wkt_ref[...], preferred_element_type=jnp.float32)
         + bk_ref[...]).astype(jnp.bfloat16)
    v = jnp.dot(toks, wvt_ref[...], preferred_element_type=jnp.float32) + bv_ref[...]
    qb = q_ref[...].astype(jnp.bfloat16)
    row = lax.broadcasted_iota(jnp.int32, (D, NH), 0)
    colh = lax.broadcasted_iota(jnp.int32, (D, NH), 1)
    ind = (row // HD == colh).astype(jnp.bfloat16)
    indt = (lax.broadcasted_iota(jnp.int32, (NH, D), 1) // HD
            == lax.broadcasted_iota(jnp.int32, (NH, D), 0)).astype(jnp.float32)
    qk = (qb[:, None, :] * k.reshape(QB, K, D)).reshape(TB, D)
    s = jnp.dot(qk, ind, preferred_element_type=jnp.float32) * (1.0 / SCALE)
    s3 = s.reshape(QB, K, NH)
    m3 = mask_ref[...][:, :, None]
    s3 = s3 * m3
    mx = jnp.max(s3, axis=1, keepdims=True)
    p = jnp.exp(s3 - mx)
    denom = jnp.sum(p, axis=1, keepdims=True)
    pv = (p * m3).reshape(TB, NH)
    wexp = jnp.dot(pv, indt, preferred_element_type=jnp.float32)
    osum = jnp.sum((wexp * v).reshape(QB, K, D), axis=1)
    dexp = jnp.dot(denom.reshape(QB, NH), indt,
                   preferred_element_type=jnp.float32)
    attn = osum / dexp
    out_ref[...] = (
        jnp.dot(attn, wot_ref[...], preferred_element_type=jnp.float32)
        + bo_ref[...]
    )


def _attention(gathered, q, maskf, wkt, wvt, wot, bk2, bv2, bo2):
    grid = (NQ // QB,)
    return pl.pallas_call(
        _attn_kernel,
        grid=grid,
        in_specs=[
            pl.BlockSpec((TB, D), lambda i: (i, 0)),
            pl.BlockSpec((QB, D), lambda i: (i, 0)),
            pl.BlockSpec((QB, K), lambda i: (i, 0)),
            pl.BlockSpec((D, D), lambda i: (0, 0)),
            pl.BlockSpec((D, D), lambda i: (0, 0)),
            pl.BlockSpec((D, D), lambda i: (0, 0)),
            pl.BlockSpec((1, D), lambda i: (0, 0)),
            pl.BlockSpec((1, D), lambda i: (0, 0)),
            pl.BlockSpec((1, D), lambda i: (0, 0)),
        ],
        out_specs=pl.BlockSpec((QB, D), lambda i: (i, 0)),
        out_shape=jax.ShapeDtypeStruct((NQ, D), jnp.float32),
    )(gathered, q, maskf, wkt, wvt, wot, bk2, bv2, bo2)


def kernel(query, time_center, freq_center, value, spatial_shapes,
           level_start_index, Wq, bq, Wk, bk, Wv, bv, Wo, bo):
    q2 = query.reshape(NQ, D)
    tc2 = time_center.reshape(NQ, 1)
    fc2 = freq_center.reshape(NQ, 1)
    value_bf = value.reshape(TOTAL, D).astype(jnp.bfloat16)
    value_pk = jax.lax.bitcast_convert_type(
        value_bf.reshape(TOTAL, DW, 2), jnp.int32)
    q, idx, maskf = _prep(q2, tc2, fc2, Wq.T, bq.reshape(1, D))
    gathered_pk = _sc_gather(value_pk, idx.reshape(ROWS))
    gathered = jax.lax.bitcast_convert_type(
        gathered_pk, jnp.bfloat16).reshape(ROWS, D)
    out = _attention(gathered, q, maskf,
                     Wk.T.astype(jnp.bfloat16), Wv.T.astype(jnp.bfloat16),
                     Wo.T, bk.reshape(1, D), bv.reshape(1, D),
                     bo.reshape(1, D))
    return out.reshape(1, NQ, D)


# speedup vs baseline: 4.3566x; 4.3566x over previous
import functools
import math

import jax
import jax.numpy as jnp
from jax import lax
from jax.experimental import pallas as pl
from jax.experimental.pallas import tpu as pltpu
from jax.experimental.pallas import tpu_sc as plsc

D = 256
NH = 8
HD = 32
NQ = 1024
K = 128
TOTAL = 43520
LVL_W0 = 1024
LVL_H0 = 32
SCALE = math.sqrt(HD)

SC_CORES = 2
SC_SUBCORES = 16
NW = SC_CORES * SC_SUBCORES
ROWS = NQ * K
RPW = ROWS // NW
CH = 128
NCH = RPW // CH
NBUF = 3


def _prep_kernel(q_ref, tc_ref, fc_ref, wqt_ref, bq_ref,
                 qout_ref, idx_ref, mask_ref):
    qout_ref[...] = (
        jnp.dot(q_ref[...], wqt_ref[...], preferred_element_type=jnp.float32)
        + bq_ref[...]
    )
    tc = tc_ref[...]
    fc = fc_ref[...]
    col = lax.broadcasted_iota(jnp.int32, (NQ, K), 1)
    lvl = col // 32
    within = col % 32
    t_off = within % 8 - 4
    f_off = within // 8 - 2
    w_i = jnp.int32(LVL_W0) >> lvl
    h_i = jnp.int32(LVL_H0) >> lvl
    lsi = jnp.where(lvl == 0, 0,
          jnp.where(lvl == 1, 32768,
          jnp.where(lvl == 2, 40960, 43008)))
    tpx = jnp.round(tc * w_i.astype(jnp.float32) - 0.5).astype(jnp.int32)
    fpx = jnp.round(fc * h_i.astype(jnp.float32) - 0.5).astype(jnp.int32)
    tt = tpx + t_off
    ff = fpx + f_off
    oob = (tt < 0) | (tt >= w_i) | (ff < 0) | (ff >= h_i)
    ttc = jnp.clip(tt, 0, w_i - 1)
    ffc = jnp.clip(ff, 0, h_i - 1)
    idx_ref[...] = lsi + ffc * w_i + ttc
    mask_ref[...] = jnp.where(oob, 0.0, 1.0)


def _prep(q2, tc2, fc2, wqt, bq2):
    return pl.pallas_call(
        _prep_kernel,
        out_shape=[
            jax.ShapeDtypeStruct((NQ, D), jnp.float32),
            jax.ShapeDtypeStruct((NQ, K), jnp.int32),
            jax.ShapeDtypeStruct((NQ, K), jnp.float32),
        ],
    )(q2, tc2, fc2, wqt, bq2)


def _sc_gather(value2, idx_flat):
    mesh = plsc.VectorSubcoreMesh(core_axis_name="c", subcore_axis_name="s")

    @functools.partial(
        pl.kernel,
        mesh=mesh,
        out_type=jax.ShapeDtypeStruct((ROWS, D), jnp.float32),
        scratch_types=[
            pltpu.VMEM((RPW,), jnp.int32),
            [pltpu.VMEM((CH, D), jnp.float32) for _ in range(NBUF)],
            [pltpu.SemaphoreType.DMA for _ in range(NBUF)],
            [pltpu.SemaphoreType.DMA for _ in range(NBUF)],
        ],
    )
    def gather(value_hbm, idx_hbm, out_hbm, idx_all, bufs, gsems, ssems):
        wid = lax.axis_index("s") * SC_CORES + lax.axis_index("c")
        base = wid * RPW
        pltpu.sync_copy(idx_hbm.at[pl.ds(base, RPW)], idx_all)

        def fire_gather(g, b):
            return pltpu.async_copy(
                value_hbm.at[idx_all.at[pl.ds(g * CH, CH)]], bufs[b], gsems[b])

        def fire_scatter(g, b):
            return pltpu.async_copy(
                bufs[b], out_hbm.at[pl.ds(base + g * CH, CH)], ssems[b])

        gd = [None] * NBUF
        sd = [None] * NBUF
        for g in range(NBUF):
            gd[g] = fire_gather(g, g)
        for g in range(NCH):
            b = g % NBUF
            gd[b].wait()
            sd[b] = fire_scatter(g, b)
            nxt = g + NBUF
            if nxt < NCH:
                sd[b].wait()
                gd[b] = fire_gather(nxt, b)
        for g in range(NCH - min(NBUF, NCH), NCH):
            sd[g % NBUF].wait()

    return gather(value2, idx_flat)


QB = 64
TB = QB * K


def _attn_kernel(toks_ref, q_ref, mask_ref, wkt_ref, wvt_ref, wot_ref,
                 bk_ref, bv_ref, bo_ref, out_ref):
    toks = toks_ref[...].astype(jnp.bfloat16)
    k = (jnp.dot(toks, wkt_ref[...], preferred_element_type=jnp.float32)
         + bk_ref[...]).astype(jnp.bfloat16)
    v = jnp.dot(toks, wvt_ref[...], preferred_element_type=jnp.float32) + bv_ref[...]
    qb = q_ref[...].astype(jnp.bfloat16)
    row = lax.broadcasted_iota(jnp.int32, (D, NH), 0)
    colh = lax.broadcasted_iota(jnp.int32, (D, NH), 1)
    ind = (row // HD == colh).astype(jnp.bfloat16)
    indt = (lax.broadcasted_iota(jnp.int32, (NH, D), 1) // HD
            == lax.broadcasted_iota(jnp.int32, (NH, D), 0)).astype(jnp.float32)
    qk = (qb[:, None, :] * k.reshape(QB, K, D)).reshape(TB, D)
    s = jnp.dot(qk, ind, preferred_element_type=jnp.float32) * (1.0 / SCALE)
    s3 = s.reshape(QB, K, NH)
    m3 = mask_ref[...][:, :, None]
    s3 = s3 * m3
    mx = jnp.max(s3, axis=1, keepdims=True)
    p = jnp.exp(s3 - mx)
    denom = jnp.sum(p, axis=1, keepdims=True)
    pv = (p * m3).reshape(TB, NH)
    wexp = jnp.dot(pv, indt, preferred_element_type=jnp.float32)
    osum = jnp.sum((wexp * v).reshape(QB, K, D), axis=1)
    dexp = jnp.dot(denom.reshape(QB, NH), indt,
                   preferred_element_type=jnp.float32)
    attn = osum / dexp
    out_ref[...] = (
        jnp.dot(attn, wot_ref[...], preferred_element_type=jnp.float32)
        + bo_ref[...]
    )


def _attention(gathered, q, maskf, wkt, wvt, wot, bk2, bv2, bo2):
    grid = (NQ // QB,)
    return pl.pallas_call(
        _attn_kernel,
        grid=grid,
        in_specs=[
            pl.BlockSpec((TB, D), lambda i: (i, 0)),
            pl.BlockSpec((QB, D), lambda i: (i, 0)),
            pl.BlockSpec((QB, K), lambda i: (i, 0)),
            pl.BlockSpec((D, D), lambda i: (0, 0)),
            pl.BlockSpec((D, D), lambda i: (0, 0)),
            pl.BlockSpec((D, D), lambda i: (0, 0)),
            pl.BlockSpec((1, D), lambda i: (0, 0)),
            pl.BlockSpec((1, D), lambda i: (0, 0)),
            pl.BlockSpec((1, D), lambda i: (0, 0)),
        ],
        out_specs=pl.BlockSpec((QB, D), lambda i: (i, 0)),
        out_shape=jax.ShapeDtypeStruct((NQ, D), jnp.float32),
    )(gathered, q, maskf, wkt, wvt, wot, bk2, bv2, bo2)


def kernel(query, time_center, freq_center, value, spatial_shapes,
           level_start_index, Wq, bq, Wk, bk, Wv, bv, Wo, bo):
    q2 = query.reshape(NQ, D)
    tc2 = time_center.reshape(NQ, 1)
    fc2 = freq_center.reshape(NQ, 1)
    value2 = value.reshape(TOTAL, D)
    q, idx, maskf = _prep(q2, tc2, fc2, Wq.T, bq.reshape(1, D))
    gathered = _sc_gather(value2, idx.reshape(ROWS))
    out = _attention(gathered, q, maskf,
                     Wk.T.astype(jnp.bfloat16), Wv.T.astype(jnp.bfloat16),
                     Wo.T, bk.reshape(1, D), bv.reshape(1, D),
                     bo.reshape(1, D))
    return out.reshape(1, NQ, D)


# speedup vs baseline: 4.6675x; 1.0714x over previous
import functools
import math

import jax
import jax.numpy as jnp
from jax import lax
from jax.experimental import pallas as pl
from jax.experimental.pallas import tpu as pltpu
from jax.experimental.pallas import tpu_sc as plsc

D = 256
NH = 8
HD = 32
NQ = 1024
K = 128
TOTAL = 43520
LVL_W0 = 1024
LVL_H0 = 32
SCALE = math.sqrt(HD)

SC_CORES = 2
SC_SUBCORES = 16
NW = SC_CORES * SC_SUBCORES
ROWS = NQ * K
RPW = ROWS // NW
CH = 128
NCH = RPW // CH
NBUF = 3
NSPLIT = 2


def _prep_kernel(q_ref, tc_ref, fc_ref, wqt_ref, bq_ref,
                 qout_ref, idx_ref, mask_ref):
    qout_ref[...] = (
        jnp.dot(q_ref[...], wqt_ref[...], preferred_element_type=jnp.float32)
        + bq_ref[...]
    )
    tc = tc_ref[...]
    fc = fc_ref[...]
    col = lax.broadcasted_iota(jnp.int32, (NQ, K), 1)
    lvl = col // 32
    within = col % 32
    t_off = within % 8 - 4
    f_off = within // 8 - 2
    w_i = jnp.int32(LVL_W0) >> lvl
    h_i = jnp.int32(LVL_H0) >> lvl
    lsi = jnp.where(lvl == 0, 0,
          jnp.where(lvl == 1, 32768,
          jnp.where(lvl == 2, 40960, 43008)))
    tpx = jnp.round(tc * w_i.astype(jnp.float32) - 0.5).astype(jnp.int32)
    fpx = jnp.round(fc * h_i.astype(jnp.float32) - 0.5).astype(jnp.int32)
    tt = tpx + t_off
    ff = fpx + f_off
    oob = (tt < 0) | (tt >= w_i) | (ff < 0) | (ff >= h_i)
    ttc = jnp.clip(tt, 0, w_i - 1)
    ffc = jnp.clip(ff, 0, h_i - 1)
    idx_ref[...] = lsi + ffc * w_i + ttc
    mask_ref[...] = jnp.where(oob, 0.0, 1.0)


def _prep(q2, tc2, fc2, wqt, bq2):
    return pl.pallas_call(
        _prep_kernel,
        out_shape=[
            jax.ShapeDtypeStruct((NQ, D), jnp.float32),
            jax.ShapeDtypeStruct((NQ, K), jnp.int32),
            jax.ShapeDtypeStruct((NQ, K), jnp.float32),
        ],
    )(q2, tc2, fc2, wqt, bq2)


def _sc_gather(value2, idx_flat, nrows):
    mesh = plsc.VectorSubcoreMesh(core_axis_name="c", subcore_axis_name="s")
    rpw = nrows // NW
    nch = rpw // CH

    @functools.partial(
        pl.kernel,
        mesh=mesh,
        out_type=jax.ShapeDtypeStruct((nrows, D), jnp.float32),
        scratch_types=[
            pltpu.VMEM((rpw,), jnp.int32),
            [pltpu.VMEM((CH, D), jnp.float32) for _ in range(NBUF)],
            [pltpu.SemaphoreType.DMA for _ in range(NBUF)],
            [pltpu.SemaphoreType.DMA for _ in range(NBUF)],
        ],
    )
    def gather(value_hbm, idx_hbm, out_hbm, idx_all, bufs, gsems, ssems):
        wid = lax.axis_index("s") * SC_CORES + lax.axis_index("c")
        base = wid * rpw
        pltpu.sync_copy(idx_hbm.at[pl.ds(base, rpw)], idx_all)

        def fire_gather(g, b):
            return pltpu.async_copy(
                value_hbm.at[idx_all.at[pl.ds(g * CH, CH)]], bufs[b], gsems[b])

        def fire_scatter(g, b):
            return pltpu.async_copy(
                bufs[b], out_hbm.at[pl.ds(base + g * CH, CH)], ssems[b])

        gd = [None] * NBUF
        sd = [None] * NBUF
        for g in range(NBUF):
            gd[g] = fire_gather(g, g)
        for g in range(nch):
            b = g % NBUF
            gd[b].wait()
            sd[b] = fire_scatter(g, b)
            nxt = g + NBUF
            if nxt < nch:
                sd[b].wait()
                gd[b] = fire_gather(nxt, b)
        for g in range(nch - min(NBUF, nch), nch):
            sd[g % NBUF].wait()

    return gather(value2, idx_flat)


QB = 64
TB = QB * K


def _attn_kernel(toks_ref, q_ref, mask_ref, wkt_ref, wvt_ref, wot_ref,
                 bk_ref, bv_ref, bo_ref, out_ref):
    toks = toks_ref[...].astype(jnp.bfloat16)
    k = (jnp.dot(toks, wkt_ref[...], preferred_element_type=jnp.float32)
         + bk_ref[...]).astype(jnp.bfloat16)
    v = jnp.dot(toks, wvt_ref[...], preferred_element_type=jnp.float32) + bv_ref[...]
    qb = q_ref[...].astype(jnp.bfloat16)
    row = lax.broadcasted_iota(jnp.int32, (D, NH), 0)
    colh = lax.broadcasted_iota(jnp.int32, (D, NH), 1)
    ind = (row // HD == colh).astype(jnp.bfloat16)
    indt = (lax.broadcasted_iota(jnp.int32, (NH, D), 1) // HD
            == lax.broadcasted_iota(jnp.int32, (NH, D), 0)).astype(jnp.float32)
    qk = (qb[:, None, :] * k.reshape(QB, K, D)).reshape(TB, D)
    s = jnp.dot(qk, ind, preferred_element_type=jnp.float32) * (1.0 / SCALE)
    s3 = s.reshape(QB, K, NH)
    m3 = mask_ref[...][:, :, None]
    s3 = s3 * m3
    mx = jnp.max(s3, axis=1, keepdims=True)
    p = jnp.exp(s3 - mx)
    denom = jnp.sum(p, axis=1, keepdims=True)
    pv = (p * m3).reshape(TB, NH)
    wexp = jnp.dot(pv, indt, preferred_element_type=jnp.float32)
    osum = jnp.sum((wexp * v).reshape(QB, K, D), axis=1)
    dexp = jnp.dot(denom.reshape(QB, NH), indt,
                   preferred_element_type=jnp.float32)
    attn = osum / dexp
    out_ref[...] = (
        jnp.dot(attn, wot_ref[...], preferred_element_type=jnp.float32)
        + bo_ref[...]
    )


def _attention(gathered, q, maskf, wkt, wvt, wot, bk2, bv2, bo2):
    nq = q.shape[0]
    grid = (nq // QB,)
    return pl.pallas_call(
        _attn_kernel,
        grid=grid,
        in_specs=[
            pl.BlockSpec((TB, D), lambda i: (i, 0)),
            pl.BlockSpec((QB, D), lambda i: (i, 0)),
            pl.BlockSpec((QB, K), lambda i: (i, 0)),
            pl.BlockSpec((D, D), lambda i: (0, 0)),
            pl.BlockSpec((D, D), lambda i: (0, 0)),
            pl.BlockSpec((D, D), lambda i: (0, 0)),
            pl.BlockSpec((1, D), lambda i: (0, 0)),
            pl.BlockSpec((1, D), lambda i: (0, 0)),
            pl.BlockSpec((1, D), lambda i: (0, 0)),
        ],
        out_specs=pl.BlockSpec((QB, D), lambda i: (i, 0)),
        out_shape=jax.ShapeDtypeStruct((nq, D), jnp.float32),
    )(gathered, q, maskf, wkt, wvt, wot, bk2, bv2, bo2)


def kernel(query, time_center, freq_center, value, spatial_shapes,
           level_start_index, Wq, bq, Wk, bk, Wv, bv, Wo, bo):
    q2 = query.reshape(NQ, D)
    tc2 = time_center.reshape(NQ, 1)
    fc2 = freq_center.reshape(NQ, 1)
    value2 = value.reshape(TOTAL, D)
    q, idx, maskf = _prep(q2, tc2, fc2, Wq.T, bq.reshape(1, D))
    wkt = Wk.T.astype(jnp.bfloat16)
    wvt = Wv.T.astype(jnp.bfloat16)
    wot = Wo.T
    bk2, bv2, bo2 = bk.reshape(1, D), bv.reshape(1, D), bo.reshape(1, D)
    nh = NQ // NSPLIT
    outs = []
    for s0 in range(NSPLIT):
        lo = s0 * nh
        g_h = _sc_gather(value2, idx[lo:lo + nh].reshape(nh * K), nh * K)
        outs.append(_attention(g_h, q[lo:lo + nh], maskf[lo:lo + nh],
                               wkt, wvt, wot, bk2, bv2, bo2))
    out = jnp.concatenate(outs, axis=0)
    return out.reshape(1, NQ, D)


# speedup vs baseline: 4.8559x; 1.0404x over previous
import functools
import math

import jax
import jax.numpy as jnp
from jax import lax
from jax.experimental import pallas as pl
from jax.experimental.pallas import tpu as pltpu
from jax.experimental.pallas import tpu_sc as plsc

D = 256
NH = 8
HD = 32
NQ = 1024
K = 128
TOTAL = 43520
LVL_W0 = 1024
LVL_H0 = 32
SCALE = math.sqrt(HD)

SC_CORES = 2
SC_SUBCORES = 16
NW = SC_CORES * SC_SUBCORES
ROWS = NQ * K
RPW = ROWS // NW
CH = 128
NCH = RPW // CH
NBUF = 3
NSPLIT = 4


def _prep_kernel(q_ref, tc_ref, fc_ref, wqt_ref, bq_ref,
                 qout_ref, idx_ref, mask_ref):
    qout_ref[...] = (
        jnp.dot(q_ref[...], wqt_ref[...], preferred_element_type=jnp.float32)
        + bq_ref[...]
    )
    tc = tc_ref[...]
    fc = fc_ref[...]
    col = lax.broadcasted_iota(jnp.int32, (NQ, K), 1)
    lvl = col // 32
    within = col % 32
    t_off = within % 8 - 4
    f_off = within // 8 - 2
    w_i = jnp.int32(LVL_W0) >> lvl
    h_i = jnp.int32(LVL_H0) >> lvl
    lsi = jnp.where(lvl == 0, 0,
          jnp.where(lvl == 1, 32768,
          jnp.where(lvl == 2, 40960, 43008)))
    tpx = jnp.round(tc * w_i.astype(jnp.float32) - 0.5).astype(jnp.int32)
    fpx = jnp.round(fc * h_i.astype(jnp.float32) - 0.5).astype(jnp.int32)
    tt = tpx + t_off
    ff = fpx + f_off
    oob = (tt < 0) | (tt >= w_i) | (ff < 0) | (ff >= h_i)
    ttc = jnp.clip(tt, 0, w_i - 1)
    ffc = jnp.clip(ff, 0, h_i - 1)
    idx_ref[...] = lsi + ffc * w_i + ttc
    mask_ref[...] = jnp.where(oob, 0.0, 1.0)


def _prep(q2, tc2, fc2, wqt, bq2):
    return pl.pallas_call(
        _prep_kernel,
        out_shape=[
            jax.ShapeDtypeStruct((NQ, D), jnp.float32),
            jax.ShapeDtypeStruct((NQ, K), jnp.int32),
            jax.ShapeDtypeStruct((NQ, K), jnp.float32),
        ],
    )(q2, tc2, fc2, wqt, bq2)


def _sc_gather(value2, idx_flat, nrows):
    mesh = plsc.VectorSubcoreMesh(core_axis_name="c", subcore_axis_name="s")
    rpw = nrows // NW
    nch = rpw // CH

    @functools.partial(
        pl.kernel,
        mesh=mesh,
        out_type=jax.ShapeDtypeStruct((nrows, D), jnp.float32),
        scratch_types=[
            pltpu.VMEM((rpw,), jnp.int32),
            [pltpu.VMEM((CH, D), jnp.float32) for _ in range(NBUF)],
            [pltpu.SemaphoreType.DMA for _ in range(NBUF)],
            [pltpu.SemaphoreType.DMA for _ in range(NBUF)],
        ],
    )
    def gather(value_hbm, idx_hbm, out_hbm, idx_all, bufs, gsems, ssems):
        wid = lax.axis_index("s") * SC_CORES + lax.axis_index("c")
        base = wid * rpw
        pltpu.sync_copy(idx_hbm.at[pl.ds(base, rpw)], idx_all)

        def fire_gather(g, b):
            return pltpu.async_copy(
                value_hbm.at[idx_all.at[pl.ds(g * CH, CH)]], bufs[b], gsems[b])

        def fire_scatter(g, b):
            return pltpu.async_copy(
                bufs[b], out_hbm.at[pl.ds(base + g * CH, CH)], ssems[b])

        gd = [None] * NBUF
        sd = [None] * NBUF
        for g in range(NBUF):
            gd[g] = fire_gather(g, g)
        for g in range(nch):
            b = g % NBUF
            gd[b].wait()
            sd[b] = fire_scatter(g, b)
            nxt = g + NBUF
            if nxt < nch:
                sd[b].wait()
                gd[b] = fire_gather(nxt, b)
        for g in range(nch - min(NBUF, nch), nch):
            sd[g % NBUF].wait()

    return gather(value2, idx_flat)


QB = 64
TB = QB * K


def _attn_kernel(toks_ref, q_ref, mask_ref, wkt_ref, wvt_ref, wot_ref,
                 bk_ref, bv_ref, bo_ref, out_ref):
    toks = toks_ref[...].astype(jnp.bfloat16)
    k = (jnp.dot(toks, wkt_ref[...], preferred_element_type=jnp.float32)
         + bk_ref[...]).astype(jnp.bfloat16)
    v = jnp.dot(toks, wvt_ref[...], preferred_element_type=jnp.float32) + bv_ref[...]
    qb = q_ref[...].astype(jnp.bfloat16)
    row = lax.broadcasted_iota(jnp.int32, (D, NH), 0)
    colh = lax.broadcasted_iota(jnp.int32, (D, NH), 1)
    ind = (row // HD == colh).astype(jnp.bfloat16)
    indt = (lax.broadcasted_iota(jnp.int32, (NH, D), 1) // HD
            == lax.broadcasted_iota(jnp.int32, (NH, D), 0)).astype(jnp.float32)
    qk = (qb[:, None, :] * k.reshape(QB, K, D)).reshape(TB, D)
    s = jnp.dot(qk, ind, preferred_element_type=jnp.float32) * (1.0 / SCALE)
    s3 = s.reshape(QB, K, NH)
    m3 = mask_ref[...][:, :, None]
    s3 = s3 * m3
    mx = jnp.max(s3, axis=1, keepdims=True)
    p = jnp.exp(s3 - mx)
    denom = jnp.sum(p, axis=1, keepdims=True)
    pv = (p * m3).reshape(TB, NH)
    wexp = jnp.dot(pv, indt, preferred_element_type=jnp.float32)
    osum = jnp.sum((wexp * v).reshape(QB, K, D), axis=1)
    dexp = jnp.dot(denom.reshape(QB, NH), indt,
                   preferred_element_type=jnp.float32)
    attn = osum / dexp
    out_ref[...] = (
        jnp.dot(attn, wot_ref[...], preferred_element_type=jnp.float32)
        + bo_ref[...]
    )


def _attention(gathered, q, maskf, wkt, wvt, wot, bk2, bv2, bo2):
    nq = q.shape[0]
    grid = (nq // QB,)
    return pl.pallas_call(
        _attn_kernel,
        grid=grid,
        in_specs=[
            pl.BlockSpec((TB, D), lambda i: (i, 0)),
            pl.BlockSpec((QB, D), lambda i: (i, 0)),
            pl.BlockSpec((QB, K), lambda i: (i, 0)),
            pl.BlockSpec((D, D), lambda i: (0, 0)),
            pl.BlockSpec((D, D), lambda i: (0, 0)),
            pl.BlockSpec((D, D), lambda i: (0, 0)),
            pl.BlockSpec((1, D), lambda i: (0, 0)),
            pl.BlockSpec((1, D), lambda i: (0, 0)),
            pl.BlockSpec((1, D), lambda i: (0, 0)),
        ],
        out_specs=pl.BlockSpec((QB, D), lambda i: (i, 0)),
        out_shape=jax.ShapeDtypeStruct((nq, D), jnp.float32),
    )(gathered, q, maskf, wkt, wvt, wot, bk2, bv2, bo2)


def kernel(query, time_center, freq_center, value, spatial_shapes,
           level_start_index, Wq, bq, Wk, bk, Wv, bv, Wo, bo):
    q2 = query.reshape(NQ, D)
    tc2 = time_center.reshape(NQ, 1)
    fc2 = freq_center.reshape(NQ, 1)
    value2 = value.reshape(TOTAL, D)
    q, idx, maskf = _prep(q2, tc2, fc2, Wq.T, bq.reshape(1, D))
    wkt = Wk.T.astype(jnp.bfloat16)
    wvt = Wv.T.astype(jnp.bfloat16)
    wot = Wo.T
    bk2, bv2, bo2 = bk.reshape(1, D), bv.reshape(1, D), bo.reshape(1, D)
    nh = NQ // NSPLIT
    outs = []
    for s0 in range(NSPLIT):
        lo = s0 * nh
        g_h = _sc_gather(value2, idx[lo:lo + nh].reshape(nh * K), nh * K)
        outs.append(_attention(g_h, q[lo:lo + nh], maskf[lo:lo + nh],
                               wkt, wvt, wot, bk2, bv2, bo2))
    out = jnp.concatenate(outs, axis=0)
    return out.reshape(1, NQ, D)
